# baseline (speedup 1.0000x reference)
import jax
import jax.numpy as jnp
from jax import lax
from jax.experimental import pallas as pl
from jax.experimental.pallas import tpu as pltpu
from jax.experimental.pallas import tpu_sc as plsc

BATCH = 16384
EMBED_DIM = 32
NUM_ROWS = 1000000
_NGRP = EMBED_DIM // 2

_info = plsc.get_sparse_core_info()
_NC, _NS = _info.num_cores, _info.num_subcores
_NW = _NC * _NS
_B_PER_W = BATCH // _NW

_PADROW = 1 << 20
_CH = 1 << 16
_NCHUNK = _PADROW // _CH


def _detile_body(tlo_blk, thi_blk, flat_ref, w_v, sems):
    i = pl.program_id(0)
    b = i % 2

    @pl.when(i > 0)
    def _wait_prev():
        for g in range(_NGRP):
            pltpu.make_async_copy(
                w_v.at[1 - b, g],
                flat_ref.at[pl.ds(g * _PADROW + (i - 1) * _CH, _CH)],
                sems.at[1 - b, g],
            ).wait()

    lo = lax.bitcast_convert_type(
        tlo_blk[...].astype(jnp.bfloat16), jnp.uint16).astype(jnp.uint32)
    hi = lax.bitcast_convert_type(
        thi_blk[...].astype(jnp.bfloat16), jnp.uint16).astype(jnp.uint32)
    w_v[b] = lo | (hi << 16)
    for g in range(_NGRP):
        pltpu.make_async_copy(
            w_v.at[b, g], flat_ref.at[pl.ds(g * _PADROW + i * _CH, _CH)],
            sems.at[b, g],
        ).start()

    @pl.when(i == _NCHUNK - 1)
    def _wait_last():
        for g in range(_NGRP):
            pltpu.make_async_copy(
                w_v.at[b, g], flat_ref.at[pl.ds(g * _PADROW + i * _CH, _CH)],
                sems.at[b, g],
            ).wait()


def _tc_detile(t):
    return pl.pallas_call(
        _detile_body,
        grid=(_NCHUNK,),
        in_specs=[pl.BlockSpec((_NGRP, _CH), lambda i: (0, i)),
                  pl.BlockSpec((_NGRP, _CH), lambda i: (1, i))],
        out_specs=pl.BlockSpec(memory_space=pl.ANY),
        out_shape=jax.ShapeDtypeStruct((_NGRP * _PADROW,), jnp.uint32),
        scratch_shapes=[
            pltpu.VMEM((2, _NGRP, _CH), jnp.uint32),
            pltpu.SemaphoreType.DMA((2, _NGRP)),
        ],
    )(t, t)


_SC_LANES = 7 * _CH
_SC_ITEM = 2048
_SC_PER_W = _SC_LANES // _SC_ITEM // _NW


def _sc_detile_body(t_hbm, flat_hbm, lo_v, hi_v, out_v):
    wid = lax.axis_index("s") * _NC + lax.axis_index("c")

    def per_item(k, carry):
        o = (k * _NW + wid) * _SC_ITEM
        pltpu.sync_copy(t_hbm.at[pl.ds(0, _NGRP), pl.ds(o, _SC_ITEM)], lo_v)
        pltpu.sync_copy(t_hbm.at[pl.ds(_NGRP, _NGRP), pl.ds(o, _SC_ITEM)],
                        hi_v)

        def per_vec(j, c2):
            for g in range(_NGRP):
                a = lo_v[g, pl.ds(16 * j, 16)]
                b = hi_v[g, pl.ds(16 * j, 16)]
                w = plsc.bitcast(
                    plsc.pack(a, b, format=plsc.PackFormat.INTERLEAVED),
                    jnp.uint32)
                out_v[g, pl.ds(16 * j, 16)] = w
            return c2

        lax.fori_loop(0, _SC_ITEM // 16, per_vec, 0)
        for g in range(_NGRP):
            pltpu.sync_copy(out_v.at[g],
                            flat_hbm.at[pl.ds(g * _PADROW + o, _SC_ITEM)])
        return carry

    lax.fori_loop(0, _SC_PER_W, per_item, 0)


def _sc_detile(t):
    mesh = plsc.VectorSubcoreMesh(core_axis_name="c", subcore_axis_name="s")
    fn = pl.kernel(
        _sc_detile_body,
        mesh=mesh,
        compiler_params=pltpu.CompilerParams(use_tc_tiling_on_sc=True,
                                            needs_layout_passes=False),
        out_type=jax.ShapeDtypeStruct((_NGRP * _PADROW,), jnp.uint32),
        scratch_types=[
            pltpu.VMEM((_NGRP, _SC_ITEM), jnp.float32),
            pltpu.VMEM((_NGRP, _SC_ITEM), jnp.float32),
            pltpu.VMEM((_NGRP, _SC_ITEM), jnp.uint32),
        ],
    )
    return fn(t)


def _tc_detile_rest(t, flat_part):
    return pl.pallas_call(
        _detile_rest_body,
        grid=(_NCHUNK - 7,),
        in_specs=[pl.BlockSpec((_NGRP, _CH), lambda i: (0, i + 7)),
                  pl.BlockSpec((_NGRP, _CH), lambda i: (1, i + 7)),
                  pl.BlockSpec(memory_space=pl.ANY)],
        out_specs=pl.BlockSpec(memory_space=pl.ANY),
        out_shape=jax.ShapeDtypeStruct((_NGRP * _PADROW,), jnp.uint32),
        scratch_shapes=[
            pltpu.VMEM((2, _NGRP, _CH), jnp.uint32),
            pltpu.SemaphoreType.DMA((2, _NGRP)),
        ],
        input_output_aliases={2: 0},
    )(t, t, flat_part)


def _detile_rest_body(tlo_blk, thi_blk, part_ref, flat_ref, w_v, sems):
    del part_ref
    i = pl.program_id(0) + 7
    b = i % 2

    @pl.when(i > 7)
    def _wait_prev():
        for g in range(_NGRP):
            pltpu.make_async_copy(
                w_v.at[1 - b, g],
                flat_ref.at[pl.ds(g * _PADROW + (i - 1) * _CH, _CH)],
                sems.at[1 - b, g],
            ).wait()

    lo = lax.bitcast_convert_type(
        tlo_blk[...].astype(jnp.bfloat16), jnp.uint16).astype(jnp.uint32)
    hi = lax.bitcast_convert_type(
        thi_blk[...].astype(jnp.bfloat16), jnp.uint16).astype(jnp.uint32)
    w_v[b] = lo | (hi << 16)
    for g in range(_NGRP):
        pltpu.make_async_copy(
            w_v.at[b, g], flat_ref.at[pl.ds(g * _PADROW + i * _CH, _CH)],
            sems.at[b, g],
        ).start()

    @pl.when(i == _NCHUNK - 1)
    def _wait_last():
        for g in range(_NGRP):
            pltpu.make_async_copy(
                w_v.at[b, g], flat_ref.at[pl.ds(g * _PADROW + i * _CH, _CH)],
                sems.at[b, g],
            ).wait()


def _build_flat_idx(base_idx_v, flat_idx_v):
    def per_grp(g, _):
        off = g * _PADROW
        for k in range(_B_PER_W // 16):
            chunk = base_idx_v[pl.ds(16 * k, 16)]
            flat_idx_v[pl.ds(g * _B_PER_W + 16 * k, 16)] = chunk + off
        return 0
    lax.fori_loop(0, _NGRP, per_grp, 0)


def _gather_body(idx_hbm, flat_hbm, out_hbm, bidx_v, fidx_v, rows_v, sem):
    wid = lax.axis_index("s") * _NC + lax.axis_index("c")
    base = wid * _B_PER_W
    pltpu.sync_copy(idx_hbm.at[pl.ds(base, _B_PER_W)], bidx_v)
    _build_flat_idx(bidx_v, fidx_v)
    pltpu.async_copy(flat_hbm.at[fidx_v], rows_v, sem).wait()
    for g in range(_NGRP):
        pltpu.sync_copy(rows_v.at[pl.ds(g * _B_PER_W, _B_PER_W)],
                        out_hbm.at[pl.ds(g * BATCH + base, _B_PER_W)])


def _sc_gather(idx, flat):
    mesh = plsc.VectorSubcoreMesh(core_axis_name="c", subcore_axis_name="s")
    fn = pl.kernel(
        _gather_body,
        mesh=mesh,
        compiler_params=pltpu.CompilerParams(use_tc_tiling_on_sc=False),
        out_type=jax.ShapeDtypeStruct((_NGRP * BATCH,), jnp.uint32),
        scratch_types=[
            pltpu.VMEM((_B_PER_W,), jnp.int32),
            pltpu.VMEM((_NGRP * _B_PER_W,), jnp.int32),
            pltpu.VMEM((_NGRP * _B_PER_W,), jnp.uint32),
            pltpu.SemaphoreType.DMA,
        ],
    )
    return fn(idx, flat)


def _unpack(x_u32):
    lo = lax.bitcast_convert_type(x_u32 << 16, jnp.float32)
    hi = lax.bitcast_convert_type(x_u32 & jnp.uint32(0xFFFF0000), jnp.float32)
    return lo, hi


def _dense_body(ug_ref, ig_ref, wu_ref, bu_ref, wi_ref, bi_ref, out_ref):
    dn = (((1,), (0,)), ((), ()))
    u_lo, u_hi = _unpack(ug_ref[...])
    i_lo, i_hi = _unpack(ig_ref[...])
    wu = wu_ref[...]
    wi = wi_ref[...]
    uv = (
        lax.dot_general(wu[:, :_NGRP], u_lo, dimension_numbers=dn,
                        preferred_element_type=jnp.float32,
                        precision=lax.Precision.HIGHEST)
        + lax.dot_general(wu[:, _NGRP:], u_hi, dimension_numbers=dn,
                          preferred_element_type=jnp.float32,
                          precision=lax.Precision.HIGHEST)
        + bu_ref[...][:, None, None]
    )
    iv = (
        lax.dot_general(wi[:, :_NGRP], i_lo, dimension_numbers=dn,
                        preferred_element_type=jnp.float32,
                        precision=lax.Precision.HIGHEST)
        + lax.dot_general(wi[:, _NGRP:], i_hi, dimension_numbers=dn,
                          preferred_element_type=jnp.float32,
                          precision=lax.Precision.HIGHEST)
        + bi_ref[...][:, None, None]
    )
    out_ref[...] = jnp.sum(uv * iv, axis=0)


_TC_BLOCK = 32


def _tc_dense(ug3, ig3, W_user, b_user, W_item, b_item):
    nblk = 128 // _TC_BLOCK
    return pl.pallas_call(
        _dense_body,
        grid=(nblk,),
        in_specs=[
            pl.BlockSpec((_NGRP, _TC_BLOCK, 128), lambda i: (0, i, 0)),
            pl.BlockSpec((_NGRP, _TC_BLOCK, 128), lambda i: (0, i, 0)),
            pl.BlockSpec((EMBED_DIM, EMBED_DIM), lambda i: (0, 0)),
            pl.BlockSpec((EMBED_DIM,), lambda i: (0,)),
            pl.BlockSpec((EMBED_DIM, EMBED_DIM), lambda i: (0, 0)),
            pl.BlockSpec((EMBED_DIM,), lambda i: (0,)),
        ],
        out_specs=pl.BlockSpec((_TC_BLOCK, 128), lambda i: (i, 0)),
        out_shape=jax.ShapeDtypeStruct((128, 128), jnp.float32),
    )(ug3, ig3, W_user, b_user, W_item, b_item)


@jax.jit
def kernel(users, items, user_embedding, item_embedding,
           W_user, b_user, W_item, b_item):
    users = users.astype(jnp.int32)
    items = items.astype(jnp.int32)
    iflat_a = _sc_detile(item_embedding.T)
    uflat = _tc_detile(user_embedding.T)
    ug = _sc_gather(users, uflat).reshape(_NGRP, 128, 128)
    iflat = _tc_detile_rest(item_embedding.T, iflat_a)
    ig = _sc_gather(items, iflat).reshape(_NGRP, 128, 128)
    out = _tc_dense(ug, ig, W_user, b_user, W_item, b_item)
    return out.reshape(BATCH)


# speedup vs baseline: 1.0551x; 1.0551x over previous
import jax
import jax.numpy as jnp
from jax import lax
from jax.experimental import pallas as pl
from jax.experimental.pallas import tpu as pltpu
from jax.experimental.pallas import tpu_sc as plsc

BATCH = 16384
EMBED_DIM = 32
NUM_ROWS = 1000000
_NGRP = EMBED_DIM // 2

_info = plsc.get_sparse_core_info()
_NC, _NS = _info.num_cores, _info.num_subcores
_NW = _NC * _NS
_B_PER_W = BATCH // _NW

_PADROW = 1 << 20
_CH = 1 << 16
_NCHUNK = _PADROW // _CH


def _detile_body(tlo_blk, thi_blk, flat_ref, w_v, sems):
    i = pl.program_id(0)
    b = i % 2

    @pl.when(i > 0)
    def _wait_prev():
        for g in range(_NGRP):
            pltpu.make_async_copy(
                w_v.at[1 - b, g],
                flat_ref.at[pl.ds(g * _PADROW + (i - 1) * _CH, _CH)],
                sems.at[1 - b, g],
            ).wait()

    lo = lax.bitcast_convert_type(
        tlo_blk[...].astype(jnp.bfloat16), jnp.uint16).astype(jnp.uint32)
    hi = lax.bitcast_convert_type(
        thi_blk[...].astype(jnp.bfloat16), jnp.uint16).astype(jnp.uint32)
    w_v[b] = lo | (hi << 16)
    for g in range(_NGRP):
        pltpu.make_async_copy(
            w_v.at[b, g], flat_ref.at[pl.ds(g * _PADROW + i * _CH, _CH)],
            sems.at[b, g],
        ).start()

    @pl.when(i == _NCHUNK - 1)
    def _wait_last():
        for g in range(_NGRP):
            pltpu.make_async_copy(
                w_v.at[b, g], flat_ref.at[pl.ds(g * _PADROW + i * _CH, _CH)],
                sems.at[b, g],
            ).wait()


def _tc_detile(t):
    return pl.pallas_call(
        _detile_body,
        grid=(_NCHUNK,),
        in_specs=[pl.BlockSpec((_NGRP, _CH), lambda i: (0, i)),
                  pl.BlockSpec((_NGRP, _CH), lambda i: (1, i))],
        out_specs=pl.BlockSpec(memory_space=pl.ANY),
        out_shape=jax.ShapeDtypeStruct((_NGRP * _PADROW,), jnp.uint32),
        scratch_shapes=[
            pltpu.VMEM((2, _NGRP, _CH), jnp.uint32),
            pltpu.SemaphoreType.DMA((2, _NGRP)),
        ],
    )(t, t)


_SC_LANES = 4 * _CH
_SC_ITEM = 2048
_SC_PER_W = _SC_LANES // _SC_ITEM // _NW


def _sc_detile_body(t_hbm, flat_hbm, lo_v, hi_v, out_v):
    wid = lax.axis_index("s") * _NC + lax.axis_index("c")

    def per_item(k, carry):
        o = (k * _NW + wid) * _SC_ITEM
        pltpu.sync_copy(t_hbm.at[pl.ds(0, _NGRP), pl.ds(o, _SC_ITEM)], lo_v)
        pltpu.sync_copy(t_hbm.at[pl.ds(_NGRP, _NGRP), pl.ds(o, _SC_ITEM)],
                        hi_v)

        def per_vec(j, c2):
            for g in range(_NGRP):
                a = lo_v[g, pl.ds(16 * j, 16)]
                b = hi_v[g, pl.ds(16 * j, 16)]
                w = plsc.bitcast(
                    plsc.pack(a, b, format=plsc.PackFormat.INTERLEAVED),
                    jnp.uint32)
                out_v[g, pl.ds(16 * j, 16)] = w
            return c2

        lax.fori_loop(0, _SC_ITEM // 16, per_vec, 0)
        for g in range(_NGRP):
            pltpu.sync_copy(out_v.at[g],
                            flat_hbm.at[pl.ds(g * _PADROW + o, _SC_ITEM)])
        return carry

    lax.fori_loop(0, _SC_PER_W, per_item, 0)


def _sc_detile(t):
    mesh = plsc.VectorSubcoreMesh(core_axis_name="c", subcore_axis_name="s")
    fn = pl.kernel(
        _sc_detile_body,
        mesh=mesh,
        compiler_params=pltpu.CompilerParams(use_tc_tiling_on_sc=True,
                                            needs_layout_passes=False),
        out_type=jax.ShapeDtypeStruct((_NGRP * _PADROW,), jnp.uint32),
        scratch_types=[
            pltpu.VMEM((_NGRP, _SC_ITEM), jnp.float32),
            pltpu.VMEM((_NGRP, _SC_ITEM), jnp.float32),
            pltpu.VMEM((_NGRP, _SC_ITEM), jnp.uint32),
        ],
    )
    return fn(t)


def _tc_detile_rest(t, flat_part):
    return pl.pallas_call(
        _detile_rest_body,
        grid=(_NCHUNK - 4,),
        in_specs=[pl.BlockSpec((_NGRP, _CH), lambda i: (0, i + 4)),
                  pl.BlockSpec((_NGRP, _CH), lambda i: (1, i + 4)),
                  pl.BlockSpec(memory_space=pl.ANY)],
        out_specs=pl.BlockSpec(memory_space=pl.ANY),
        out_shape=jax.ShapeDtypeStruct((_NGRP * _PADROW,), jnp.uint32),
        scratch_shapes=[
            pltpu.VMEM((2, _NGRP, _CH), jnp.uint32),
            pltpu.SemaphoreType.DMA((2, _NGRP)),
        ],
        input_output_aliases={2: 0},
    )(t, t, flat_part)


def _detile_rest_body(tlo_blk, thi_blk, part_ref, flat_ref, w_v, sems):
    del part_ref
    i = pl.program_id(0) + 4
    b = i % 2

    @pl.when(i > 4)
    def _wait_prev():
        for g in range(_NGRP):
            pltpu.make_async_copy(
                w_v.at[1 - b, g],
                flat_ref.at[pl.ds(g * _PADROW + (i - 1) * _CH, _CH)],
                sems.at[1 - b, g],
            ).wait()

    lo = lax.bitcast_convert_type(
        tlo_blk[...].astype(jnp.bfloat16), jnp.uint16).astype(jnp.uint32)
    hi = lax.bitcast_convert_type(
        thi_blk[...].astype(jnp.bfloat16), jnp.uint16).astype(jnp.uint32)
    w_v[b] = lo | (hi << 16)
    for g in range(_NGRP):
        pltpu.make_async_copy(
            w_v.at[b, g], flat_ref.at[pl.ds(g * _PADROW + i * _CH, _CH)],
            sems.at[b, g],
        ).start()

    @pl.when(i == _NCHUNK - 1)
    def _wait_last():
        for g in range(_NGRP):
            pltpu.make_async_copy(
                w_v.at[b, g], flat_ref.at[pl.ds(g * _PADROW + i * _CH, _CH)],
                sems.at[b, g],
            ).wait()


def _build_flat_idx(base_idx_v, flat_idx_v):
    def per_grp(g, _):
        off = g * _PADROW
        for k in range(_B_PER_W // 16):
            chunk = base_idx_v[pl.ds(16 * k, 16)]
            flat_idx_v[pl.ds(g * _B_PER_W + 16 * k, 16)] = chunk + off
        return 0
    lax.fori_loop(0, _NGRP, per_grp, 0)


def _gather_body(idx_hbm, flat_hbm, out_hbm, bidx_v, fidx_v, rows_v, sem):
    wid = lax.axis_index("s") * _NC + lax.axis_index("c")
    base = wid * _B_PER_W
    pltpu.sync_copy(idx_hbm.at[pl.ds(base, _B_PER_W)], bidx_v)
    _build_flat_idx(bidx_v, fidx_v)
    pltpu.async_copy(flat_hbm.at[fidx_v], rows_v, sem).wait()
    for g in range(_NGRP):
        pltpu.sync_copy(rows_v.at[pl.ds(g * _B_PER_W, _B_PER_W)],
                        out_hbm.at[pl.ds(g * BATCH + base, _B_PER_W)])


def _sc_gather(idx, flat):
    mesh = plsc.VectorSubcoreMesh(core_axis_name="c", subcore_axis_name="s")
    fn = pl.kernel(
        _gather_body,
        mesh=mesh,
        compiler_params=pltpu.CompilerParams(use_tc_tiling_on_sc=False),
        out_type=jax.ShapeDtypeStruct((_NGRP * BATCH,), jnp.uint32),
        scratch_types=[
            pltpu.VMEM((_B_PER_W,), jnp.int32),
            pltpu.VMEM((_NGRP * _B_PER_W,), jnp.int32),
            pltpu.VMEM((_NGRP * _B_PER_W,), jnp.uint32),
            pltpu.SemaphoreType.DMA,
        ],
    )
    return fn(idx, flat)


def _unpack(x_u32):
    lo = lax.bitcast_convert_type(x_u32 << 16, jnp.float32)
    hi = lax.bitcast_convert_type(x_u32 & jnp.uint32(0xFFFF0000), jnp.float32)
    return lo, hi


def _dense_body(ug_ref, ig_ref, wu_ref, bu_ref, wi_ref, bi_ref, out_ref):
    dn = (((1,), (0,)), ((), ()))
    u_lo, u_hi = _unpack(ug_ref[...])
    i_lo, i_hi = _unpack(ig_ref[...])
    wu = wu_ref[...]
    wi = wi_ref[...]
    uv = (
        lax.dot_general(wu[:, :_NGRP], u_lo, dimension_numbers=dn,
                        preferred_element_type=jnp.float32,
                        precision=lax.Precision.HIGHEST)
        + lax.dot_general(wu[:, _NGRP:], u_hi, dimension_numbers=dn,
                          preferred_element_type=jnp.float32,
                          precision=lax.Precision.HIGHEST)
        + bu_ref[...][:, None, None]
    )
    iv = (
        lax.dot_general(wi[:, :_NGRP], i_lo, dimension_numbers=dn,
                        preferred_element_type=jnp.float32,
                        precision=lax.Precision.HIGHEST)
        + lax.dot_general(wi[:, _NGRP:], i_hi, dimension_numbers=dn,
                          preferred_element_type=jnp.float32,
                          precision=lax.Precision.HIGHEST)
        + bi_ref[...][:, None, None]
    )
    out_ref[...] = jnp.sum(uv * iv, axis=0)


_TC_BLOCK = 32


def _tc_dense(ug3, ig3, W_user, b_user, W_item, b_item):
    nblk = 128 // _TC_BLOCK
    return pl.pallas_call(
        _dense_body,
        grid=(nblk,),
        in_specs=[
            pl.BlockSpec((_NGRP, _TC_BLOCK, 128), lambda i: (0, i, 0)),
            pl.BlockSpec((_NGRP, _TC_BLOCK, 128), lambda i: (0, i, 0)),
            pl.BlockSpec((EMBED_DIM, EMBED_DIM), lambda i: (0, 0)),
            pl.BlockSpec((EMBED_DIM,), lambda i: (0,)),
            pl.BlockSpec((EMBED_DIM, EMBED_DIM), lambda i: (0, 0)),
            pl.BlockSpec((EMBED_DIM,), lambda i: (0,)),
        ],
        out_specs=pl.BlockSpec((_TC_BLOCK, 128), lambda i: (i, 0)),
        out_shape=jax.ShapeDtypeStruct((128, 128), jnp.float32),
    )(ug3, ig3, W_user, b_user, W_item, b_item)


@jax.jit
def kernel(users, items, user_embedding, item_embedding,
           W_user, b_user, W_item, b_item):
    users = users.astype(jnp.int32)
    items = items.astype(jnp.int32)
    iflat_a = _sc_detile(item_embedding.T)
    uflat = _tc_detile(user_embedding.T)
    ug = _sc_gather(users, uflat).reshape(_NGRP, 128, 128)
    iflat = _tc_detile_rest(item_embedding.T, iflat_a)
    ig = _sc_gather(items, iflat).reshape(_NGRP, 128, 128)
    out = _tc_dense(ug, ig, W_user, b_user, W_item, b_item)
    return out.reshape(BATCH)


# speedup vs baseline: 1.0696x; 1.0137x over previous
import jax
import jax.numpy as jnp
from jax import lax
from jax.experimental import pallas as pl
from jax.experimental.pallas import tpu as pltpu
from jax.experimental.pallas import tpu_sc as plsc

BATCH = 16384
EMBED_DIM = 32
NUM_ROWS = 1000000
_NGRP = EMBED_DIM // 2

_info = plsc.get_sparse_core_info()
_NC, _NS = _info.num_cores, _info.num_subcores
_NW = _NC * _NS
_B_PER_W = BATCH // _NW

_PADROW = 1 << 20
_CH = 1 << 16
_NCHUNK = _PADROW // _CH


def _detile_body(tlo_blk, thi_blk, flat_ref, w_v, sems):
    i = pl.program_id(0)
    b = i % 2

    @pl.when(i > 0)
    def _wait_prev():
        for g in range(_NGRP):
            pltpu.make_async_copy(
                w_v.at[1 - b, g],
                flat_ref.at[pl.ds(g * _PADROW + (i - 1) * _CH, _CH)],
                sems.at[1 - b, g],
            ).wait()

    lo = lax.bitcast_convert_type(
        tlo_blk[...].astype(jnp.bfloat16), jnp.uint16).astype(jnp.uint32)
    hi = lax.bitcast_convert_type(
        thi_blk[...].astype(jnp.bfloat16), jnp.uint16).astype(jnp.uint32)
    w_v[b] = lo | (hi << 16)
    for g in range(_NGRP):
        pltpu.make_async_copy(
            w_v.at[b, g], flat_ref.at[pl.ds(g * _PADROW + i * _CH, _CH)],
            sems.at[b, g],
        ).start()

    @pl.when(i == _NCHUNK - 1)
    def _wait_last():
        for g in range(_NGRP):
            pltpu.make_async_copy(
                w_v.at[b, g], flat_ref.at[pl.ds(g * _PADROW + i * _CH, _CH)],
                sems.at[b, g],
            ).wait()


def _tc_detile(t):
    return pl.pallas_call(
        _detile_body,
        grid=(_NCHUNK,),
        in_specs=[pl.BlockSpec((_NGRP, _CH), lambda i: (0, i)),
                  pl.BlockSpec((_NGRP, _CH), lambda i: (1, i))],
        out_specs=pl.BlockSpec(memory_space=pl.ANY),
        out_shape=jax.ShapeDtypeStruct((_NGRP * _PADROW,), jnp.uint32),
        scratch_shapes=[
            pltpu.VMEM((2, _NGRP, _CH), jnp.uint32),
            pltpu.SemaphoreType.DMA((2, _NGRP)),
        ],
    )(t, t)


def _build_flat_idx(base_idx_v, flat_idx_v):
    def per_grp(g, _):
        off = g * _PADROW
        for k in range(_B_PER_W // 16):
            chunk = base_idx_v[pl.ds(16 * k, 16)]
            flat_idx_v[pl.ds(g * _B_PER_W + 16 * k, 16)] = chunk + off
        return 0
    lax.fori_loop(0, _NGRP, per_grp, 0)


def _gather_body(idx_hbm, flat_hbm, out_hbm, bidx_v, fidx_v, rows_v, sem):
    wid = lax.axis_index("s") * _NC + lax.axis_index("c")
    base = wid * _B_PER_W
    pltpu.sync_copy(idx_hbm.at[pl.ds(base, _B_PER_W)], bidx_v)
    _build_flat_idx(bidx_v, fidx_v)
    pltpu.async_copy(flat_hbm.at[fidx_v], rows_v, sem).wait()
    for g in range(_NGRP):
        pltpu.sync_copy(rows_v.at[pl.ds(g * _B_PER_W, _B_PER_W)],
                        out_hbm.at[pl.ds(g * BATCH + base, _B_PER_W)])


def _sc_gather(idx, flat):
    mesh = plsc.VectorSubcoreMesh(core_axis_name="c", subcore_axis_name="s")
    fn = pl.kernel(
        _gather_body,
        mesh=mesh,
        compiler_params=pltpu.CompilerParams(use_tc_tiling_on_sc=False),
        out_type=jax.ShapeDtypeStruct((_NGRP * BATCH,), jnp.uint32),
        scratch_types=[
            pltpu.VMEM((_B_PER_W,), jnp.int32),
            pltpu.VMEM((_NGRP * _B_PER_W,), jnp.int32),
            pltpu.VMEM((_NGRP * _B_PER_W,), jnp.uint32),
            pltpu.SemaphoreType.DMA,
        ],
    )
    return fn(idx, flat)


def _unpack(x_u32):
    lo = lax.bitcast_convert_type(x_u32 << 16, jnp.float32)
    hi = lax.bitcast_convert_type(x_u32 & jnp.uint32(0xFFFF0000), jnp.float32)
    return lo, hi


def _dense_body(ug_ref, ig_ref, wu_ref, bu_ref, wi_ref, bi_ref, out_ref):
    dn = (((1,), (0,)), ((), ()))
    u_lo, u_hi = _unpack(ug_ref[...])
    i_lo, i_hi = _unpack(ig_ref[...])
    wu = wu_ref[...]
    wi = wi_ref[...]
    uv = (
        lax.dot_general(wu[:, :_NGRP], u_lo, dimension_numbers=dn,
                        preferred_element_type=jnp.float32,
                        precision=lax.Precision.HIGHEST)
        + lax.dot_general(wu[:, _NGRP:], u_hi, dimension_numbers=dn,
                          preferred_element_type=jnp.float32,
                          precision=lax.Precision.HIGHEST)
        + bu_ref[...][:, None, None]
    )
    iv = (
        lax.dot_general(wi[:, :_NGRP], i_lo, dimension_numbers=dn,
                        preferred_element_type=jnp.float32,
                        precision=lax.Precision.HIGHEST)
        + lax.dot_general(wi[:, _NGRP:], i_hi, dimension_numbers=dn,
                          preferred_element_type=jnp.float32,
                          precision=lax.Precision.HIGHEST)
        + bi_ref[...][:, None, None]
    )
    out_ref[...] = jnp.sum(uv * iv, axis=0)


_TC_BLOCK = 32


def _tc_dense(ug3, ig3, W_user, b_user, W_item, b_item):
    nblk = 128 // _TC_BLOCK
    return pl.pallas_call(
        _dense_body,
        grid=(nblk,),
        in_specs=[
            pl.BlockSpec((_NGRP, _TC_BLOCK, 128), lambda i: (0, i, 0)),
            pl.BlockSpec((_NGRP, _TC_BLOCK, 128), lambda i: (0, i, 0)),
            pl.BlockSpec((EMBED_DIM, EMBED_DIM), lambda i: (0, 0)),
            pl.BlockSpec((EMBED_DIM,), lambda i: (0,)),
            pl.BlockSpec((EMBED_DIM, EMBED_DIM), lambda i: (0, 0)),
            pl.BlockSpec((EMBED_DIM,), lambda i: (0,)),
        ],
        out_specs=pl.BlockSpec((_TC_BLOCK, 128), lambda i: (i, 0)),
        out_shape=jax.ShapeDtypeStruct((128, 128), jnp.float32),
    )(ug3, ig3, W_user, b_user, W_item, b_item)


@jax.jit
def kernel(users, items, user_embedding, item_embedding,
           W_user, b_user, W_item, b_item):
    users = users.astype(jnp.int32)
    items = items.astype(jnp.int32)
    uflat = _tc_detile(user_embedding.T)
    ug = _sc_gather(users, uflat).reshape(_NGRP, 128, 128)
    iflat = _tc_detile(item_embedding.T)
    ig = _sc_gather(items, iflat).reshape(_NGRP, 128, 128)
    out = _tc_dense(ug, ig, W_user, b_user, W_item, b_item)
    return out.reshape(BATCH)


# speedup vs baseline: 1.0781x; 1.0079x over previous
import jax
import jax.numpy as jnp
from jax import lax
from jax.experimental import pallas as pl
from jax.experimental.pallas import tpu as pltpu
from jax.experimental.pallas import tpu_sc as plsc

BATCH = 16384
EMBED_DIM = 32
NUM_ROWS = 1000000
_NGRP = EMBED_DIM // 2

_info = plsc.get_sparse_core_info()
_NC, _NS = _info.num_cores, _info.num_subcores
_NW = _NC * _NS
_B_PER_W = BATCH // _NW

_PADROW = 1 << 20
_CH = 1 << 17
_NCHUNK = _PADROW // _CH


def _detile_body(tlo_blk, thi_blk, flat_ref, w_v, sems):
    i = pl.program_id(0)
    b = i % 2

    @pl.when(i > 0)
    def _wait_prev():
        for g in range(_NGRP):
            pltpu.make_async_copy(
                w_v.at[1 - b, g],
                flat_ref.at[pl.ds(g * _PADROW + (i - 1) * _CH, _CH)],
                sems.at[1 - b, g],
            ).wait()

    lo = lax.bitcast_convert_type(
        tlo_blk[...].astype(jnp.bfloat16), jnp.uint16).astype(jnp.uint32)
    hi = lax.bitcast_convert_type(
        thi_blk[...].astype(jnp.bfloat16), jnp.uint16).astype(jnp.uint32)
    w_v[b] = lo | (hi << 16)
    for g in range(_NGRP):
        pltpu.make_async_copy(
            w_v.at[b, g], flat_ref.at[pl.ds(g * _PADROW + i * _CH, _CH)],
            sems.at[b, g],
        ).start()

    @pl.when(i == _NCHUNK - 1)
    def _wait_last():
        for g in range(_NGRP):
            pltpu.make_async_copy(
                w_v.at[b, g], flat_ref.at[pl.ds(g * _PADROW + i * _CH, _CH)],
                sems.at[b, g],
            ).wait()


def _tc_detile(t):
    return pl.pallas_call(
        _detile_body,
        grid=(_NCHUNK,),
        in_specs=[pl.BlockSpec((_NGRP, _CH), lambda i: (0, i)),
                  pl.BlockSpec((_NGRP, _CH), lambda i: (1, i))],
        out_specs=pl.BlockSpec(memory_space=pl.ANY),
        out_shape=jax.ShapeDtypeStruct((_NGRP * _PADROW,), jnp.uint32),
        scratch_shapes=[
            pltpu.VMEM((2, _NGRP, _CH), jnp.uint32),
            pltpu.SemaphoreType.DMA((2, _NGRP)),
        ],
    )(t, t)


def _build_flat_idx(base_idx_v, flat_idx_v):
    def per_grp(g, _):
        off = g * _PADROW
        for k in range(_B_PER_W // 16):
            chunk = base_idx_v[pl.ds(16 * k, 16)]
            flat_idx_v[pl.ds(g * _B_PER_W + 16 * k, 16)] = chunk + off
        return 0
    lax.fori_loop(0, _NGRP, per_grp, 0)


def _gather_body(idx_hbm, flat_hbm, out_hbm, bidx_v, fidx_v, rows_v, sem):
    wid = lax.axis_index("s") * _NC + lax.axis_index("c")
    base = wid * _B_PER_W
    pltpu.sync_copy(idx_hbm.at[pl.ds(base, _B_PER_W)], bidx_v)
    _build_flat_idx(bidx_v, fidx_v)
    pltpu.async_copy(flat_hbm.at[fidx_v], rows_v, sem).wait()
    for g in range(_NGRP):
        pltpu.sync_copy(rows_v.at[pl.ds(g * _B_PER_W, _B_PER_W)],
                        out_hbm.at[pl.ds(g * BATCH + base, _B_PER_W)])


def _sc_gather(idx, flat):
    mesh = plsc.VectorSubcoreMesh(core_axis_name="c", subcore_axis_name="s")
    fn = pl.kernel(
        _gather_body,
        mesh=mesh,
        compiler_params=pltpu.CompilerParams(use_tc_tiling_on_sc=False),
        out_type=jax.ShapeDtypeStruct((_NGRP * BATCH,), jnp.uint32),
        scratch_types=[
            pltpu.VMEM((_B_PER_W,), jnp.int32),
            pltpu.VMEM((_NGRP * _B_PER_W,), jnp.int32),
            pltpu.VMEM((_NGRP * _B_PER_W,), jnp.uint32),
            pltpu.SemaphoreType.DMA,
        ],
    )
    return fn(idx, flat)


def _unpack(x_u32):
    lo = lax.bitcast_convert_type(x_u32 << 16, jnp.float32)
    hi = lax.bitcast_convert_type(x_u32 & jnp.uint32(0xFFFF0000), jnp.float32)
    return lo, hi


def _dense_body(ug_ref, ig_ref, wu_ref, bu_ref, wi_ref, bi_ref, out_ref):
    dn = (((1,), (0,)), ((), ()))
    u_lo, u_hi = _unpack(ug_ref[...])
    i_lo, i_hi = _unpack(ig_ref[...])
    wu = wu_ref[...]
    wi = wi_ref[...]
    uv = (
        lax.dot_general(wu[:, :_NGRP], u_lo, dimension_numbers=dn,
                        preferred_element_type=jnp.float32,
                        precision=lax.Precision.HIGHEST)
        + lax.dot_general(wu[:, _NGRP:], u_hi, dimension_numbers=dn,
                          preferred_element_type=jnp.float32,
                          precision=lax.Precision.HIGHEST)
        + bu_ref[...][:, None, None]
    )
    iv = (
        lax.dot_general(wi[:, :_NGRP], i_lo, dimension_numbers=dn,
                        preferred_element_type=jnp.float32,
                        precision=lax.Precision.HIGHEST)
        + lax.dot_general(wi[:, _NGRP:], i_hi, dimension_numbers=dn,
                          preferred_element_type=jnp.float32,
                          precision=lax.Precision.HIGHEST)
        + bi_ref[...][:, None, None]
    )
    out_ref[...] = jnp.sum(uv * iv, axis=0)


_TC_BLOCK = 32


def _tc_dense(ug3, ig3, W_user, b_user, W_item, b_item):
    nblk = 128 // _TC_BLOCK
    return pl.pallas_call(
        _dense_body,
        grid=(nblk,),
        in_specs=[
            pl.BlockSpec((_NGRP, _TC_BLOCK, 128), lambda i: (0, i, 0)),
            pl.BlockSpec((_NGRP, _TC_BLOCK, 128), lambda i: (0, i, 0)),
            pl.BlockSpec((EMBED_DIM, EMBED_DIM), lambda i: (0, 0)),
            pl.BlockSpec((EMBED_DIM,), lambda i: (0,)),
            pl.BlockSpec((EMBED_DIM, EMBED_DIM), lambda i: (0, 0)),
            pl.BlockSpec((EMBED_DIM,), lambda i: (0,)),
        ],
        out_specs=pl.BlockSpec((_TC_BLOCK, 128), lambda i: (i, 0)),
        out_shape=jax.ShapeDtypeStruct((128, 128), jnp.float32),
    )(ug3, ig3, W_user, b_user, W_item, b_item)


@jax.jit
def kernel(users, items, user_embedding, item_embedding,
           W_user, b_user, W_item, b_item):
    users = users.astype(jnp.int32)
    items = items.astype(jnp.int32)
    uflat = _tc_detile(user_embedding.T)
    ug = _sc_gather(users, uflat).reshape(_NGRP, 128, 128)
    iflat = _tc_detile(item_embedding.T)
    ig = _sc_gather(items, iflat).reshape(_NGRP, 128, 128)
    out = _tc_dense(ug, ig, W_user, b_user, W_item, b_item)
    return out.reshape(BATCH)


# speedup vs baseline: 1.0805x; 1.0022x over previous
import jax
import jax.numpy as jnp
from jax import lax
from jax.experimental import pallas as pl
from jax.experimental.pallas import tpu as pltpu
from jax.experimental.pallas import tpu_sc as plsc

BATCH = 16384
EMBED_DIM = 32
NUM_ROWS = 1000000
_NGRP = EMBED_DIM // 2

_info = plsc.get_sparse_core_info()
_NC, _NS = _info.num_cores, _info.num_subcores
_NW = _NC * _NS
_B_PER_W = BATCH // _NW

_PADROW = 1 << 20
_CH = 1 << 17
_NCHUNK = _PADROW // _CH


def _detile_body(tlo_blk, thi_blk, flat_ref, w_v, sems):
    i = pl.program_id(0)
    b = i % 2

    @pl.when(i > 0)
    def _wait_prev():
        for g in range(_NGRP):
            pltpu.make_async_copy(
                w_v.at[1 - b, g],
                flat_ref.at[pl.ds(g * _PADROW + (i - 1) * _CH, _CH)],
                sems.at[1 - b, g],
            ).wait()

    lo = lax.bitcast_convert_type(
        tlo_blk[...].astype(jnp.bfloat16), jnp.uint16).astype(jnp.uint32)
    hi = lax.bitcast_convert_type(
        thi_blk[...].astype(jnp.bfloat16), jnp.uint16).astype(jnp.uint32)
    w_v[b] = lo | (hi << 16)
    for g in range(_NGRP):
        pltpu.make_async_copy(
            w_v.at[b, g], flat_ref.at[pl.ds(g * _PADROW + i * _CH, _CH)],
            sems.at[b, g],
        ).start()

    @pl.when(i == _NCHUNK - 1)
    def _wait_last():
        for g in range(_NGRP):
            pltpu.make_async_copy(
                w_v.at[b, g], flat_ref.at[pl.ds(g * _PADROW + i * _CH, _CH)],
                sems.at[b, g],
            ).wait()


def _tc_detile(t):
    return pl.pallas_call(
        _detile_body,
        grid=(_NCHUNK,),
        in_specs=[pl.BlockSpec((_NGRP, _CH), lambda i: (0, i)),
                  pl.BlockSpec((_NGRP, _CH), lambda i: (1, i))],
        out_specs=pl.BlockSpec(memory_space=pl.ANY),
        out_shape=jax.ShapeDtypeStruct((_NGRP * _PADROW,), jnp.uint32),
        scratch_shapes=[
            pltpu.VMEM((2, _NGRP, _CH), jnp.uint32),
            pltpu.SemaphoreType.DMA((2, _NGRP)),
        ],
    )(t, t)


def _build_flat_idx(base_idx_v, flat_idx_v):
    def per_grp(g, _):
        off = g * _PADROW
        for k in range(_B_PER_W // 16):
            chunk = base_idx_v[pl.ds(16 * k, 16)]
            flat_idx_v[pl.ds(g * _B_PER_W + 16 * k, 16)] = chunk + off
        return 0
    lax.fori_loop(0, _NGRP, per_grp, 0)


def _gather_body(idx_hbm, flat_hbm, out_hbm, bidx_v, fidx_v, rows_v, sem):
    wid = lax.axis_index("s") * _NC + lax.axis_index("c")
    base = wid * _B_PER_W
    pltpu.sync_copy(idx_hbm.at[pl.ds(base, _B_PER_W)], bidx_v)
    _build_flat_idx(bidx_v, fidx_v)
    pltpu.async_copy(flat_hbm.at[fidx_v], rows_v, sem).wait()
    for g in range(_NGRP):
        pltpu.sync_copy(rows_v.at[pl.ds(g * _B_PER_W, _B_PER_W)],
                        out_hbm.at[pl.ds(g * BATCH + base, _B_PER_W)])


def _sc_gather(idx, flat):
    mesh = plsc.VectorSubcoreMesh(core_axis_name="c", subcore_axis_name="s")
    fn = pl.kernel(
        _gather_body,
        mesh=mesh,
        compiler_params=pltpu.CompilerParams(use_tc_tiling_on_sc=False),
        out_type=jax.ShapeDtypeStruct((_NGRP * BATCH,), jnp.uint32),
        scratch_types=[
            pltpu.VMEM((_B_PER_W,), jnp.int32),
            pltpu.VMEM((_NGRP * _B_PER_W,), jnp.int32),
            pltpu.VMEM((_NGRP * _B_PER_W,), jnp.uint32),
            pltpu.SemaphoreType.DMA,
        ],
    )
    return fn(idx, flat)


def _unpack(x_u32):
    lo = lax.bitcast_convert_type(x_u32 << 16, jnp.float32)
    hi = lax.bitcast_convert_type(x_u32 & jnp.uint32(0xFFFF0000), jnp.float32)
    return lo, hi


def _dense_body(ug_ref, ig_ref, wu_ref, bu_ref, wi_ref, bi_ref, out_ref):
    dn = (((1,), (0,)), ((), ()))
    u_lo, u_hi = _unpack(ug_ref[...])
    i_lo, i_hi = _unpack(ig_ref[...])
    wu = wu_ref[...]
    wi = wi_ref[...]
    uv = (
        lax.dot_general(wu[:, :_NGRP], u_lo, dimension_numbers=dn,
                        preferred_element_type=jnp.float32,
                        precision=lax.Precision.HIGHEST)
        + lax.dot_general(wu[:, _NGRP:], u_hi, dimension_numbers=dn,
                          preferred_element_type=jnp.float32,
                          precision=lax.Precision.HIGHEST)
        + bu_ref[...][:, None, None]
    )
    iv = (
        lax.dot_general(wi[:, :_NGRP], i_lo, dimension_numbers=dn,
                        preferred_element_type=jnp.float32,
                        precision=lax.Precision.HIGHEST)
        + lax.dot_general(wi[:, _NGRP:], i_hi, dimension_numbers=dn,
                          preferred_element_type=jnp.float32,
                          precision=lax.Precision.HIGHEST)
        + bi_ref[...][:, None, None]
    )
    out_ref[...] = jnp.sum(uv * iv, axis=0)


_TC_BLOCK = 128


def _tc_dense(ug3, ig3, W_user, b_user, W_item, b_item):
    nblk = 128 // _TC_BLOCK
    return pl.pallas_call(
        _dense_body,
        grid=(nblk,),
        in_specs=[
            pl.BlockSpec((_NGRP, _TC_BLOCK, 128), lambda i: (0, i, 0)),
            pl.BlockSpec((_NGRP, _TC_BLOCK, 128), lambda i: (0, i, 0)),
            pl.BlockSpec((EMBED_DIM, EMBED_DIM), lambda i: (0, 0)),
            pl.BlockSpec((EMBED_DIM,), lambda i: (0,)),
            pl.BlockSpec((EMBED_DIM, EMBED_DIM), lambda i: (0, 0)),
            pl.BlockSpec((EMBED_DIM,), lambda i: (0,)),
        ],
        out_specs=pl.BlockSpec((_TC_BLOCK, 128), lambda i: (i, 0)),
        out_shape=jax.ShapeDtypeStruct((128, 128), jnp.float32),
    )(ug3, ig3, W_user, b_user, W_item, b_item)


@jax.jit
def kernel(users, items, user_embedding, item_embedding,
           W_user, b_user, W_item, b_item):
    users = users.astype(jnp.int32)
    items = items.astype(jnp.int32)
    uflat = _tc_detile(user_embedding.T)
    ug = _sc_gather(users, uflat).reshape(_NGRP, 128, 128)
    iflat = _tc_detile(item_embedding.T)
    ig = _sc_gather(items, iflat).reshape(_NGRP, 128, 128)
    out = _tc_dense(ug, ig, W_user, b_user, W_item, b_item)
    return out.reshape(BATCH)
